# baseline (speedup 1.0000x reference)
import functools

import jax
import jax.numpy as jnp
from jax import lax
from jax.experimental import pallas as pl
from jax.experimental.pallas import tpu as pltpu
from jax.experimental.pallas import tpu_sc as plsc

_DIM = 128
_NCLS = 150
_GAMMA = 0.999
_NW = 32
_DSPLIT = 8
_DSUB = _DIM // _DSPLIT
_PSPLIT = _NW // _DSPLIT
_CPAD = 160
_ACC = _DSUB * _CPAD
_CNT_PAD = 160
_PC = 2048

_PSC = 131072
_PB = 8192
_CTC = 256


def _sc_body(spw, pix_per_batch, feat_hbm, lab_hbm, part_out, cnt_out,
             feat_v0, feat_v1, lab_v0, lab_v1, acc_v, cnt_v,
             fsem0, fsem1, lsem0, lsem1):
    c = lax.axis_index("c")
    s = lax.axis_index("s")
    wid = s * 2 + c
    dgrp = wid % _DSPLIT
    pgrp = wid // _DSPLIT
    b = pgrp // 2
    off0 = (pgrp % 2) * spw
    lab0 = b * pix_per_batch + off0
    d0 = dgrp * _DSUB
    nchunks = spw // _PC

    zeros16 = jnp.zeros((16,), jnp.float32)
    ones16 = jnp.ones((16,), jnp.float32)

    def zbody(i, _):
        acc_v[pl.ds(i * 16, 16)] = zeros16
        return 0
    lax.fori_loop(0, _ACC // 16, zbody, 0)
    for i in range(_CNT_PAD // 16):
        cnt_v[pl.ds(i * 16, 16)] = zeros16

    bufs = ((feat_v0, lab_v0, fsem0, lsem0), (feat_v1, lab_v1, fsem1, lsem1))

    def lab_src(g):
        return lab_hbm.at[pl.ds(lab0 + g * _PC, _PC)]

    def feat_src(g):
        return feat_hbm.at[b, pl.ds(d0, _DSUB), pl.ds(off0 + g * _PC, _PC)]

    def start(g, bi):
        fv, lv, fs, ls = bufs[bi]
        pltpu.async_copy(lab_src(g), lv, ls)
        pltpu.async_copy(feat_src(g), fv, fs)

    def wait(g, bi):
        fv, lv, fs, ls = bufs[bi]
        pltpu.make_async_copy(lab_src(g), lv, ls).wait()
        pltpu.make_async_copy(feat_src(g), fv, fs).wait()

    def compute(bi):
        feat_v, lab_v = bufs[bi][0], bufs[bi][1]

        def group_body(j):
            lab16 = lab_v[pl.ds(j, 16)]
            plsc.addupdate_scatter(cnt_v, [lab16], ones16)
            K = 8
            xs = [feat_v[u, pl.ds(j, 16)] for u in range(K)]
            for u0 in range(0, _DSUB, K):
                nxt = u0 + K
                xs_next = ([feat_v[nxt + u, pl.ds(j, 16)] for u in range(K)]
                           if nxt < _DSUB else None)
                for u in range(K):
                    plsc.addupdate_scatter(
                        acc_v, [lab16 + ((u0 + u) * _CPAD)], xs[u])
                xs = xs_next
        plsc.parallel_loop(0, _PC, 16)(group_body)

    start(0, 0)

    def pair_body(gp, _):
        g = 2 * gp
        start(g + 1, 1)
        wait(g, 0)
        compute(0)

        @pl.when(gp < nchunks // 2 - 1)
        def _():
            start(g + 2, 0)
        wait(g + 1, 1)
        compute(1)
        return 0
    lax.fori_loop(0, nchunks // 2, pair_body, 0)

    pltpu.sync_copy(acc_v, part_out.at[wid])
    pltpu.sync_copy(cnt_v, cnt_out.at[wid])


@jax.jit
def _sc_segment_sums(feat3, lab):
    B, D, P = feat3.shape
    mesh = plsc.VectorSubcoreMesh(core_axis_name="c", subcore_axis_name="s")
    body = functools.partial(_sc_body, _PSC // 2, P)
    return pl.kernel(
        body,
        out_type=(
            jax.ShapeDtypeStruct((_NW, _ACC), jnp.float32),
            jax.ShapeDtypeStruct((_NW, _CNT_PAD), jnp.float32),
        ),
        mesh=mesh,
        compiler_params=pltpu.CompilerParams(needs_layout_passes=False),
        scratch_types=[
            pltpu.VMEM((_DSUB, _PC), jnp.float32),
            pltpu.VMEM((_DSUB, _PC), jnp.float32),
            pltpu.VMEM((_PC,), jnp.int32),
            pltpu.VMEM((_PC,), jnp.int32),
            pltpu.VMEM((_ACC,), jnp.float32),
            pltpu.VMEM((_CNT_PAD,), jnp.float32),
            pltpu.SemaphoreType.DMA,
            pltpu.SemaphoreType.DMA,
            pltpu.SemaphoreType.DMA,
            pltpu.SemaphoreType.DMA,
        ],
    )(feat3, lab)


def _tc_mm_body(nb, feat_ref, lab_ref, sums_ref, cnt_ref, acc_sc, cnt_sc):
    b = pl.program_id(0)
    j = pl.program_id(1)

    @pl.when(jnp.logical_and(b == 0, j == 0))
    def _():
        acc_sc[...] = jnp.zeros_like(acc_sc)
        cnt_sc[...] = jnp.zeros_like(cnt_sc)

    lab = lab_ref[0]
    cls = lax.broadcasted_iota(jnp.int32, (_CTC, _PB), 0)
    oh = (lab == cls).astype(jnp.float32)
    f = feat_ref[0]
    acc_sc[...] += lax.dot_general(oh, f, (((1,), (1,)), ((), ())),
                                   preferred_element_type=jnp.float32)
    cnt_sc[...] += jnp.sum(oh, axis=1, keepdims=True)

    @pl.when(jnp.logical_and(b == 1, j == nb - 1))
    def _():
        sums_ref[...] = acc_sc[...]
        cnt_ref[...] = cnt_sc[...]


@jax.jit
def _tc_segment_sums(feat3, lab3):
    B, D, P = feat3.shape
    nb = (P - _PSC) // _PB
    j0 = _PSC // _PB
    return pl.pallas_call(
        functools.partial(_tc_mm_body, nb),
        grid=(B, nb),
        in_specs=[
            pl.BlockSpec((1, D, _PB), lambda b, j: (b, 0, j0 + j)),
            pl.BlockSpec((1, 1, _PB), lambda b, j: (b, 0, j0 + j)),
        ],
        out_specs=[
            pl.BlockSpec((_CTC, D), lambda b, j: (0, 0)),
            pl.BlockSpec((_CTC, 1), lambda b, j: (0, 0)),
        ],
        out_shape=(
            jax.ShapeDtypeStruct((_CTC, D), jnp.float32),
            jax.ShapeDtypeStruct((_CTC, 1), jnp.float32),
        ),
        scratch_shapes=[
            pltpu.VMEM((_CTC, D), jnp.float32),
            pltpu.VMEM((_CTC, 1), jnp.float32),
        ],
        compiler_params=pltpu.CompilerParams(
            dimension_semantics=("arbitrary", "arbitrary")),
    )(feat3, lab3)


def _tc_epilogue_body(part_ref, cnt_ref, tsum_ref, tcnt_ref,
                      proto_ref, uc_ref, outp_ref, outc_ref):
    part = part_ref[...]
    blocks = []
    for dg in range(_DSPLIT):
        acc = part[dg] + part[_DSPLIT + dg] + part[2 * _DSPLIT + dg] \
            + part[3 * _DSPLIT + dg]
        blocks.append(acc)
    sums_t = jnp.concatenate(blocks, axis=0)
    sums = jnp.transpose(sums_t)[:_NCLS, :] + tsum_ref[:_NCLS, :]
    counts = (jnp.sum(cnt_ref[...], axis=1, keepdims=True) * (1.0 / _DSPLIT)
              + tcnt_ref[:_NCLS, :])
    present = counts > 0.0
    proto = sums / jnp.maximum(counts, 1.0)
    nrm = jnp.sqrt(jnp.sum(proto * proto, axis=1, keepdims=True))
    proto_n = proto / jnp.maximum(nrm, 1e-12)
    uc = uc_ref[...]
    g = jnp.where(uc == 0.0, 0.0,
                  jnp.minimum(1.0 - 1.0 / (uc + 1.0), _GAMMA))
    upd = g * proto_ref[...] + (1.0 - g) * proto_n
    newp = jnp.where(present, upd, proto_ref[...])
    nrm2 = jnp.sqrt(jnp.sum(newp * newp, axis=1, keepdims=True))
    outp_ref[...] = newp / jnp.maximum(nrm2, 1e-12)
    outc_ref[...] = uc + present.astype(jnp.float32)


@jax.jit
def _tc_epilogue(part, cnt_t, tsum, tcnt, prototypes, update_count2):
    return pl.pallas_call(
        _tc_epilogue_body,
        out_shape=(
            jax.ShapeDtypeStruct((_NCLS, _DIM), jnp.float32),
            jax.ShapeDtypeStruct((_NCLS, 1), jnp.float32),
        ),
    )(part, cnt_t, tsum, tcnt, prototypes, update_count2)


def kernel(features, labels, prototypes, update_count):
    B, D, H, W = features.shape
    feat3 = features.reshape(B, D, H * W)
    lab = labels.reshape(-1)
    part, cnt = _sc_segment_sums(feat3, lab)
    tsum, tcnt = _tc_segment_sums(feat3, labels.reshape(B, 1, H * W))
    part = part.reshape(_NW, _DSUB, _CPAD)
    cnt_t = cnt[:, :_NCLS].T
    new_proto, new_count2 = _tc_epilogue(part, cnt_t, tsum, tcnt, prototypes,
                                         update_count.reshape(_NCLS, 1))
    return new_proto, new_count2.reshape(-1)


# speedup vs baseline: 1.0575x; 1.0575x over previous
import functools

import jax
import jax.numpy as jnp
from jax import lax
from jax.experimental import pallas as pl
from jax.experimental.pallas import tpu as pltpu
from jax.experimental.pallas import tpu_sc as plsc

_DIM = 128
_NCLS = 150
_GAMMA = 0.999
_NW = 32
_DSPLIT = 8
_DSUB = _DIM // _DSPLIT
_PSPLIT = _NW // _DSPLIT
_CPAD = 160
_ACC = _DSUB * _CPAD
_CNT_PAD = 160
_PC = 2048

_PSC = 106496
_PB = 8192
_CTC = 256


def _sc_body(spw, pix_per_batch, feat_hbm, lab_hbm, part_out, cnt_out,
             feat_v0, feat_v1, lab_v0, lab_v1, acc_v, cnt_v,
             fsem0, fsem1, lsem0, lsem1):
    c = lax.axis_index("c")
    s = lax.axis_index("s")
    wid = s * 2 + c
    dgrp = wid % _DSPLIT
    pgrp = wid // _DSPLIT
    b = pgrp // 2
    off0 = (pgrp % 2) * spw
    lab0 = b * pix_per_batch + off0
    d0 = dgrp * _DSUB
    nchunks = spw // _PC

    zeros16 = jnp.zeros((16,), jnp.float32)
    ones16 = jnp.ones((16,), jnp.float32)

    def zbody(i, _):
        acc_v[pl.ds(i * 16, 16)] = zeros16
        return 0
    lax.fori_loop(0, _ACC // 16, zbody, 0)
    for i in range(_CNT_PAD // 16):
        cnt_v[pl.ds(i * 16, 16)] = zeros16

    bufs = ((feat_v0, lab_v0, fsem0, lsem0), (feat_v1, lab_v1, fsem1, lsem1))

    def lab_src(g):
        return lab_hbm.at[pl.ds(lab0 + g * _PC, _PC)]

    def feat_src(g):
        return feat_hbm.at[b, pl.ds(d0, _DSUB), pl.ds(off0 + g * _PC, _PC)]

    def start(g, bi):
        fv, lv, fs, ls = bufs[bi]
        pltpu.async_copy(lab_src(g), lv, ls)
        pltpu.async_copy(feat_src(g), fv, fs)

    def wait(g, bi):
        fv, lv, fs, ls = bufs[bi]
        pltpu.make_async_copy(lab_src(g), lv, ls).wait()
        pltpu.make_async_copy(feat_src(g), fv, fs).wait()

    def compute(bi):
        feat_v, lab_v = bufs[bi][0], bufs[bi][1]

        def group_body(j):
            lab16 = lab_v[pl.ds(j, 16)]
            plsc.addupdate_scatter(cnt_v, [lab16], ones16)
            K = 8
            xs = [feat_v[u, pl.ds(j, 16)] for u in range(K)]
            for u0 in range(0, _DSUB, K):
                nxt = u0 + K
                xs_next = ([feat_v[nxt + u, pl.ds(j, 16)] for u in range(K)]
                           if nxt < _DSUB else None)
                for u in range(K):
                    plsc.addupdate_scatter(
                        acc_v, [lab16 + ((u0 + u) * _CPAD)], xs[u])
                xs = xs_next
        plsc.parallel_loop(0, _PC, 16)(group_body)

    start(0, 0)

    def pair_body(gp, _):
        g = 2 * gp
        start(g + 1, 1)
        wait(g, 0)
        compute(0)

        @pl.when(gp < nchunks // 2 - 1)
        def _():
            start(g + 2, 0)
        wait(g + 1, 1)
        compute(1)
        return 0
    lax.fori_loop(0, nchunks // 2, pair_body, 0)

    pltpu.sync_copy(acc_v, part_out.at[wid])
    pltpu.sync_copy(cnt_v, cnt_out.at[wid])


@jax.jit
def _sc_segment_sums(feat3, lab):
    B, D, P = feat3.shape
    mesh = plsc.VectorSubcoreMesh(core_axis_name="c", subcore_axis_name="s")
    body = functools.partial(_sc_body, _PSC // 2, P)
    return pl.kernel(
        body,
        out_type=(
            jax.ShapeDtypeStruct((_NW, _ACC), jnp.float32),
            jax.ShapeDtypeStruct((_NW, _CNT_PAD), jnp.float32),
        ),
        mesh=mesh,
        compiler_params=pltpu.CompilerParams(needs_layout_passes=False),
        scratch_types=[
            pltpu.VMEM((_DSUB, _PC), jnp.float32),
            pltpu.VMEM((_DSUB, _PC), jnp.float32),
            pltpu.VMEM((_PC,), jnp.int32),
            pltpu.VMEM((_PC,), jnp.int32),
            pltpu.VMEM((_ACC,), jnp.float32),
            pltpu.VMEM((_CNT_PAD,), jnp.float32),
            pltpu.SemaphoreType.DMA,
            pltpu.SemaphoreType.DMA,
            pltpu.SemaphoreType.DMA,
            pltpu.SemaphoreType.DMA,
        ],
    )(feat3, lab)


def _tc_mm_body(nb, feat_ref, lab_ref, sums_ref, cnt_ref, acc_sc, cnt_sc):
    b = pl.program_id(0)
    j = pl.program_id(1)

    @pl.when(jnp.logical_and(b == 0, j == 0))
    def _():
        acc_sc[...] = jnp.zeros_like(acc_sc)
        cnt_sc[...] = jnp.zeros_like(cnt_sc)

    lab = lab_ref[0]
    cls = lax.broadcasted_iota(jnp.int32, (_CTC, _PB), 0)
    oh = (lab == cls).astype(jnp.float32)
    f = feat_ref[0]
    acc_sc[...] += lax.dot_general(oh, f, (((1,), (1,)), ((), ())),
                                   preferred_element_type=jnp.float32)
    cnt_sc[...] += jnp.sum(oh, axis=1, keepdims=True)

    @pl.when(jnp.logical_and(b == 1, j == nb - 1))
    def _():
        sums_ref[...] = acc_sc[...]
        cnt_ref[...] = cnt_sc[...]


@jax.jit
def _tc_segment_sums(feat3, lab3):
    B, D, P = feat3.shape
    nb = (P - _PSC) // _PB
    j0 = _PSC // _PB
    return pl.pallas_call(
        functools.partial(_tc_mm_body, nb),
        grid=(B, nb),
        in_specs=[
            pl.BlockSpec((1, D, _PB), lambda b, j: (b, 0, j0 + j)),
            pl.BlockSpec((1, 1, _PB), lambda b, j: (b, 0, j0 + j)),
        ],
        out_specs=[
            pl.BlockSpec((_CTC, D), lambda b, j: (0, 0)),
            pl.BlockSpec((_CTC, 1), lambda b, j: (0, 0)),
        ],
        out_shape=(
            jax.ShapeDtypeStruct((_CTC, D), jnp.float32),
            jax.ShapeDtypeStruct((_CTC, 1), jnp.float32),
        ),
        scratch_shapes=[
            pltpu.VMEM((_CTC, D), jnp.float32),
            pltpu.VMEM((_CTC, 1), jnp.float32),
        ],
        compiler_params=pltpu.CompilerParams(
            dimension_semantics=("arbitrary", "arbitrary")),
    )(feat3, lab3)


def _tc_epilogue_body(part_ref, cnt_ref, tsum_ref, tcnt_ref,
                      proto_ref, uc_ref, outp_ref, outc_ref):
    part = part_ref[...]
    blocks = []
    for dg in range(_DSPLIT):
        acc = part[dg] + part[_DSPLIT + dg] + part[2 * _DSPLIT + dg] \
            + part[3 * _DSPLIT + dg]
        blocks.append(acc)
    sums_t = jnp.concatenate(blocks, axis=0)
    sums = jnp.transpose(sums_t)[:_NCLS, :] + tsum_ref[:_NCLS, :]
    counts = (jnp.sum(cnt_ref[...], axis=1, keepdims=True) * (1.0 / _DSPLIT)
              + tcnt_ref[:_NCLS, :])
    present = counts > 0.0
    proto = sums / jnp.maximum(counts, 1.0)
    nrm = jnp.sqrt(jnp.sum(proto * proto, axis=1, keepdims=True))
    proto_n = proto / jnp.maximum(nrm, 1e-12)
    uc = uc_ref[...]
    g = jnp.where(uc == 0.0, 0.0,
                  jnp.minimum(1.0 - 1.0 / (uc + 1.0), _GAMMA))
    upd = g * proto_ref[...] + (1.0 - g) * proto_n
    newp = jnp.where(present, upd, proto_ref[...])
    nrm2 = jnp.sqrt(jnp.sum(newp * newp, axis=1, keepdims=True))
    outp_ref[...] = newp / jnp.maximum(nrm2, 1e-12)
    outc_ref[...] = uc + present.astype(jnp.float32)


@jax.jit
def _tc_epilogue(part, cnt_t, tsum, tcnt, prototypes, update_count2):
    return pl.pallas_call(
        _tc_epilogue_body,
        out_shape=(
            jax.ShapeDtypeStruct((_NCLS, _DIM), jnp.float32),
            jax.ShapeDtypeStruct((_NCLS, 1), jnp.float32),
        ),
    )(part, cnt_t, tsum, tcnt, prototypes, update_count2)


def kernel(features, labels, prototypes, update_count):
    B, D, H, W = features.shape
    feat3 = features.reshape(B, D, H * W)
    lab = labels.reshape(-1)
    part, cnt = _sc_segment_sums(feat3, lab)
    tsum, tcnt = _tc_segment_sums(feat3, labels.reshape(B, 1, H * W))
    part = part.reshape(_NW, _DSUB, _CPAD)
    cnt_t = cnt[:, :_NCLS].T
    new_proto, new_count2 = _tc_epilogue(part, cnt_t, tsum, tcnt, prototypes,
                                         update_count.reshape(_NCLS, 1))
    return new_proto, new_count2.reshape(-1)


# speedup vs baseline: 1.1021x; 1.0422x over previous
import functools

import jax
import jax.numpy as jnp
from jax import lax
from jax.experimental import pallas as pl
from jax.experimental.pallas import tpu as pltpu
from jax.experimental.pallas import tpu_sc as plsc

_DIM = 128
_NCLS = 150
_GAMMA = 0.999
_NW = 32
_DSPLIT = 8
_DSUB = _DIM // _DSPLIT
_PSPLIT = _NW // _DSPLIT
_CPAD = 160
_ACC = _DSUB * _CPAD
_CNT_PAD = 160
_PC = 2048

_PSC = 81920
_PB = 8192
_CTC = 256


def _sc_body(spw, pix_per_batch, feat_hbm, lab_hbm, part_out, cnt_out,
             feat_v0, feat_v1, lab_v0, lab_v1, acc_v, cnt_v,
             fsem0, fsem1, lsem0, lsem1):
    c = lax.axis_index("c")
    s = lax.axis_index("s")
    wid = s * 2 + c
    dgrp = wid % _DSPLIT
    pgrp = wid // _DSPLIT
    b = pgrp // 2
    off0 = (pgrp % 2) * spw
    lab0 = b * pix_per_batch + off0
    d0 = dgrp * _DSUB
    nchunks = spw // _PC

    zeros16 = jnp.zeros((16,), jnp.float32)
    ones16 = jnp.ones((16,), jnp.float32)

    def zbody(i, _):
        acc_v[pl.ds(i * 16, 16)] = zeros16
        return 0
    lax.fori_loop(0, _ACC // 16, zbody, 0)
    for i in range(_CNT_PAD // 16):
        cnt_v[pl.ds(i * 16, 16)] = zeros16

    bufs = ((feat_v0, lab_v0, fsem0, lsem0), (feat_v1, lab_v1, fsem1, lsem1))

    def lab_src(g):
        return lab_hbm.at[pl.ds(lab0 + g * _PC, _PC)]

    def feat_src(g):
        return feat_hbm.at[b, pl.ds(d0, _DSUB), pl.ds(off0 + g * _PC, _PC)]

    def start(g, bi):
        fv, lv, fs, ls = bufs[bi]
        pltpu.async_copy(lab_src(g), lv, ls)
        pltpu.async_copy(feat_src(g), fv, fs)

    def wait(g, bi):
        fv, lv, fs, ls = bufs[bi]
        pltpu.make_async_copy(lab_src(g), lv, ls).wait()
        pltpu.make_async_copy(feat_src(g), fv, fs).wait()

    def compute(bi):
        feat_v, lab_v = bufs[bi][0], bufs[bi][1]

        def group_body(j):
            lab16 = lab_v[pl.ds(j, 16)]
            plsc.addupdate_scatter(cnt_v, [lab16], ones16)
            K = 8
            xs = [feat_v[u, pl.ds(j, 16)] for u in range(K)]
            for u0 in range(0, _DSUB, K):
                nxt = u0 + K
                xs_next = ([feat_v[nxt + u, pl.ds(j, 16)] for u in range(K)]
                           if nxt < _DSUB else None)
                for u in range(K):
                    plsc.addupdate_scatter(
                        acc_v, [lab16 + ((u0 + u) * _CPAD)], xs[u])
                xs = xs_next
        plsc.parallel_loop(0, _PC, 16)(group_body)

    start(0, 0)

    def pair_body(gp, _):
        g = 2 * gp
        start(g + 1, 1)
        wait(g, 0)
        compute(0)

        @pl.when(gp < nchunks // 2 - 1)
        def _():
            start(g + 2, 0)
        wait(g + 1, 1)
        compute(1)
        return 0
    lax.fori_loop(0, nchunks // 2, pair_body, 0)

    pltpu.sync_copy(acc_v, part_out.at[wid])
    pltpu.sync_copy(cnt_v, cnt_out.at[wid])


@jax.jit
def _sc_segment_sums(feat3, lab):
    B, D, P = feat3.shape
    mesh = plsc.VectorSubcoreMesh(core_axis_name="c", subcore_axis_name="s")
    body = functools.partial(_sc_body, _PSC // 2, P)
    return pl.kernel(
        body,
        out_type=(
            jax.ShapeDtypeStruct((_NW, _ACC), jnp.float32),
            jax.ShapeDtypeStruct((_NW, _CNT_PAD), jnp.float32),
        ),
        mesh=mesh,
        compiler_params=pltpu.CompilerParams(needs_layout_passes=False),
        scratch_types=[
            pltpu.VMEM((_DSUB, _PC), jnp.float32),
            pltpu.VMEM((_DSUB, _PC), jnp.float32),
            pltpu.VMEM((_PC,), jnp.int32),
            pltpu.VMEM((_PC,), jnp.int32),
            pltpu.VMEM((_ACC,), jnp.float32),
            pltpu.VMEM((_CNT_PAD,), jnp.float32),
            pltpu.SemaphoreType.DMA,
            pltpu.SemaphoreType.DMA,
            pltpu.SemaphoreType.DMA,
            pltpu.SemaphoreType.DMA,
        ],
    )(feat3, lab)


def _tc_mm_body(nb, feat_ref, lab_ref, sums_ref, cnt_ref, acc_sc, cnt_sc):
    b = pl.program_id(0)
    j = pl.program_id(1)

    @pl.when(jnp.logical_and(b == 0, j == 0))
    def _():
        acc_sc[...] = jnp.zeros_like(acc_sc)
        cnt_sc[...] = jnp.zeros_like(cnt_sc)

    lab = lab_ref[0]
    cls = lax.broadcasted_iota(jnp.int32, (_CTC, _PB), 0)
    oh = (lab == cls).astype(jnp.float32)
    f = feat_ref[0]
    acc_sc[...] += lax.dot_general(oh, f, (((1,), (1,)), ((), ())),
                                   preferred_element_type=jnp.float32)
    cnt_sc[...] += jnp.sum(oh, axis=1, keepdims=True)

    @pl.when(jnp.logical_and(b == 1, j == nb - 1))
    def _():
        sums_ref[...] = acc_sc[...]
        cnt_ref[...] = cnt_sc[...]


@jax.jit
def _tc_segment_sums(feat3, lab3):
    B, D, P = feat3.shape
    nb = (P - _PSC) // _PB
    j0 = _PSC // _PB
    return pl.pallas_call(
        functools.partial(_tc_mm_body, nb),
        grid=(B, nb),
        in_specs=[
            pl.BlockSpec((1, D, _PB), lambda b, j: (b, 0, j0 + j)),
            pl.BlockSpec((1, 1, _PB), lambda b, j: (b, 0, j0 + j)),
        ],
        out_specs=[
            pl.BlockSpec((_CTC, D), lambda b, j: (0, 0)),
            pl.BlockSpec((_CTC, 1), lambda b, j: (0, 0)),
        ],
        out_shape=(
            jax.ShapeDtypeStruct((_CTC, D), jnp.float32),
            jax.ShapeDtypeStruct((_CTC, 1), jnp.float32),
        ),
        scratch_shapes=[
            pltpu.VMEM((_CTC, D), jnp.float32),
            pltpu.VMEM((_CTC, 1), jnp.float32),
        ],
        compiler_params=pltpu.CompilerParams(
            dimension_semantics=("arbitrary", "arbitrary")),
    )(feat3, lab3)


def _tc_epilogue_body(part_ref, cnt_ref, tsum_ref, tcnt_ref,
                      proto_ref, uc_ref, outp_ref, outc_ref):
    part = part_ref[...]
    blocks = []
    for dg in range(_DSPLIT):
        acc = part[dg] + part[_DSPLIT + dg] + part[2 * _DSPLIT + dg] \
            + part[3 * _DSPLIT + dg]
        blocks.append(acc)
    sums_t = jnp.concatenate(blocks, axis=0)
    sums = jnp.transpose(sums_t)[:_NCLS, :] + tsum_ref[:_NCLS, :]
    counts = (jnp.sum(cnt_ref[...], axis=1, keepdims=True) * (1.0 / _DSPLIT)
              + tcnt_ref[:_NCLS, :])
    present = counts > 0.0
    proto = sums / jnp.maximum(counts, 1.0)
    nrm = jnp.sqrt(jnp.sum(proto * proto, axis=1, keepdims=True))
    proto_n = proto / jnp.maximum(nrm, 1e-12)
    uc = uc_ref[...]
    g = jnp.where(uc == 0.0, 0.0,
                  jnp.minimum(1.0 - 1.0 / (uc + 1.0), _GAMMA))
    upd = g * proto_ref[...] + (1.0 - g) * proto_n
    newp = jnp.where(present, upd, proto_ref[...])
    nrm2 = jnp.sqrt(jnp.sum(newp * newp, axis=1, keepdims=True))
    outp_ref[...] = newp / jnp.maximum(nrm2, 1e-12)
    outc_ref[...] = uc + present.astype(jnp.float32)


@jax.jit
def _tc_epilogue(part, cnt_t, tsum, tcnt, prototypes, update_count2):
    return pl.pallas_call(
        _tc_epilogue_body,
        out_shape=(
            jax.ShapeDtypeStruct((_NCLS, _DIM), jnp.float32),
            jax.ShapeDtypeStruct((_NCLS, 1), jnp.float32),
        ),
    )(part, cnt_t, tsum, tcnt, prototypes, update_count2)


def kernel(features, labels, prototypes, update_count):
    B, D, H, W = features.shape
    feat3 = features.reshape(B, D, H * W)
    lab = labels.reshape(-1)
    part, cnt = _sc_segment_sums(feat3, lab)
    tsum, tcnt = _tc_segment_sums(feat3, labels.reshape(B, 1, H * W))
    part = part.reshape(_NW, _DSUB, _CPAD)
    cnt_t = cnt[:, :_NCLS].T
    new_proto, new_count2 = _tc_epilogue(part, cnt_t, tsum, tcnt, prototypes,
                                         update_count.reshape(_NCLS, 1))
    return new_proto, new_count2.reshape(-1)
